# Initial kernel scaffold; baseline (speedup 1.0000x reference)
import jax
import jax.numpy as jnp
from jax.experimental import pallas as pl


def kernel(text, table, W, b):
    raise NotImplementedError("write your pallas kernel here")


# speedup vs baseline: 17.1294x; 17.1294x over previous
import functools

import jax
import jax.numpy as jnp
from jax import lax
from jax.experimental import pallas as pl
from jax.experimental.pallas import tpu as pltpu
from jax.experimental.pallas import tpu_sc as plsc

B = 16384
S = 200
E = 32
NCLS = 10

NC = 2
NS = 16
NW = NC * NS
BPW = B // NW
CH = 128
NCH = BPW // CH
NBUF = 2


def _sc_embed_sum(table, text_w):
  mesh = plsc.VectorSubcoreMesh(
      core_axis_name="c", subcore_axis_name="s", num_cores=NC,
      num_subcores=NS)

  @functools.partial(
      pl.kernel,
      out_type=jax.ShapeDtypeStruct((B, E), jnp.float32),
      mesh=mesh,
      scratch_types=[
          pltpu.VMEM((S, NCH, CH), jnp.int32),
          pltpu.VMEM((BPW, E), jnp.float32),
          pltpu.SemaphoreType.DMA,
          pltpu.SemaphoreType.DMA,
      ],
      compiler_params=pltpu.CompilerParams(use_tc_tiling_on_sc=False),
  )
  def body(table_hbm, text_hbm, out_hbm, idx_v, acc_v, sem_idx, sem_g):
    wid = lax.axis_index("s") * NC + lax.axis_index("c")
    base = wid * BPW

    pltpu.async_copy(text_hbm.at[wid], idx_v, sem_idx).wait()

    zeros = jnp.zeros((16,), jnp.float32)

    def zbody(i, carry):
      acc_v[i, pl.ds(0, 16)] = zeros
      acc_v[i, pl.ds(16, 16)] = zeros
      return carry

    lax.fori_loop(0, BPW, zbody, 0, unroll=4)

    def fire(s):
      for c in range(NCH):
        pltpu.async_copy(
            table_hbm.at[idx_v.at[s, c]],
            acc_v.at[pl.ds(c * CH, CH)],
            sem_g, add=True)

    def drain_one():
      pltpu.make_async_copy(
          table_hbm.at[idx_v.at[0, 0]],
          acc_v.at[pl.ds(0, CH)], sem_g).wait()

    for j in range(NBUF):
      fire(j)

    def gbody(s, carry):
      for _ in range(NCH):
        drain_one()
      fire(s)
      return carry

    lax.fori_loop(NBUF, S, gbody, 0)
    for j in range(NBUF * NCH):
      drain_one()

    pltpu.async_copy(acc_v, out_hbm.at[pl.ds(base, BPW)], sem_idx).wait()

  return body(table, text_w)


def _tc_head(sums, w_pad, b_pad):
  BLK = 2048

  def body(x_ref, w_ref, b_ref, o_ref):
    acc = jnp.dot(x_ref[...], w_ref[...], preferred_element_type=jnp.float32)
    o_ref[...] = acc * (1.0 / S) + b_ref[...]

  return pl.pallas_call(
      body,
      grid=(B // BLK,),
      in_specs=[
          pl.BlockSpec((BLK, E), lambda i: (i, 0)),
          pl.BlockSpec((E, 128), lambda i: (0, 0)),
          pl.BlockSpec((1, 128), lambda i: (0, 0)),
      ],
      out_specs=pl.BlockSpec((BLK, 128), lambda i: (i, 0)),
      out_shape=jax.ShapeDtypeStruct((B, 128), jnp.float32),
  )(sums, w_pad, b_pad)


@jax.jit
def kernel(text, table, W, b):
  text_w = text.T.reshape(S, NW, NCH, CH).transpose(1, 0, 2, 3)
  sums = _sc_embed_sum(table, text_w)
  w_pad = jnp.pad(W, ((0, 0), (0, 128 - NCLS)))
  b_pad = jnp.pad(b, (0, 128 - NCLS)).reshape(1, 128)
  logits = _tc_head(sums, w_pad, b_pad)
  return logits[:, :NCLS]
